# baseline (speedup 1.0000x reference)
import jax
import jax.numpy as jnp
from jax import lax
from jax.experimental import pallas as pl
from jax.experimental.pallas import tpu as pltpu
from jax.experimental.pallas import tpu_sc as plsc

N = 10000
E = 320000
D = 128
H = 128
O = 64

NC = 2
NS = 16
NW = NC * NS
K = 80
NCHUNK = 128
HALF = NCHUNK // 2
EPT = NCHUNK * K
EPAD = NW * EPT
NBUF = 2
MAIN_ITERS = HALF // NBUF - 1
NF = N + 8
ZROWS = 624
ZTAIL = N - NS * ZROWS


def _sc_aggregate(w_by_src: bool, width: int):
    mesh = plsc.VectorSubcoreMesh(core_axis_name="c", subcore_axis_name="s")

    def body(feat_hbm, src_hbm, dst_hbm, wtab_hbm, zrows_hbm, zvec_hbm,
             acc_out, side_out,
             acc_sh, side_sh, sidx, didx, rows, wvec, gsem, wgsem, ssem,
             wssem):
        c = lax.axis_index("c")
        s = lax.axis_index("s")
        tid = c * NS + s

        pltpu.sync_copy(zrows_hbm.at[pl.ds(s * ZROWS, ZROWS)],
                        acc_sh.at[pl.ds(s * ZROWS, ZROWS)])

        @pl.when(s == 0)
        def _():
            pltpu.sync_copy(zrows_hbm.at[pl.ds(NS * ZROWS, ZTAIL)],
                            acc_sh.at[pl.ds(NS * ZROWS, ZTAIL)])
            pltpu.sync_copy(zvec_hbm, side_sh)

        plsc.subcore_barrier()

        def issue_gather(b, i):
            pltpu.async_copy(feat_hbm.at[sidx.at[i]], rows[b], gsem[b])
            widx = sidx if w_by_src else didx
            pltpu.async_copy(wtab_hbm.at[widx.at[i]], wvec[b], wgsem[b])

        def wait_gather(b, i):
            pltpu.make_async_copy(feat_hbm.at[sidx.at[i]], rows[b],
                                  gsem[b]).wait()
            widx = sidx if w_by_src else didx
            pltpu.make_async_copy(wtab_hbm.at[widx.at[i]], wvec[b],
                                  wgsem[b]).wait()

        def process(b, i):
            rdesc = pltpu.async_copy(rows[b], acc_sh.at[didx.at[i]], ssem[b],
                                     add=True)
            tgt = didx if w_by_src else sidx
            wdesc = pltpu.async_copy(wvec[b], side_sh.at[tgt.at[i]], wssem[b],
                                     add=True)
            rdesc.wait()
            wdesc.wait()

        for half in range(2):
            pltpu.sync_copy(src_hbm.at[tid, pl.ds(half * HALF, HALF)], sidx)
            pltpu.sync_copy(dst_hbm.at[tid, pl.ds(half * HALF, HALF)], didx)

            for b in range(NBUF):
                issue_gather(b, b)

            def step(j, carry):
                for b in range(NBUF):
                    i = j * NBUF + b
                    wait_gather(b, i)
                    process(b, i)
                    issue_gather(b, i + NBUF)
                return carry

            lax.fori_loop(0, MAIN_ITERS, step, 0)

            for b in range(NBUF):
                i = MAIN_ITERS * NBUF + b
                wait_gather(b, i)
                process(b, i)

        plsc.subcore_barrier()

        pltpu.sync_copy(acc_sh.at[pl.ds(s * ZROWS, ZROWS)],
                        acc_out.at[c, pl.ds(s * ZROWS, ZROWS)])

        @pl.when(s == 0)
        def _():
            pltpu.sync_copy(acc_sh.at[pl.ds(NS * ZROWS, ZTAIL)],
                            acc_out.at[c, pl.ds(NS * ZROWS, ZTAIL)])
            pltpu.sync_copy(side_sh, side_out.at[c])

    return pl.kernel(
        body,
        out_type=[
            jax.ShapeDtypeStruct((NC, N, width), jnp.float32),
            jax.ShapeDtypeStruct((NC, NF), jnp.float32),
        ],
        mesh=mesh,
        scratch_types=[
            pltpu.VMEM_SHARED((N, width), jnp.float32),
            pltpu.VMEM_SHARED((NF,), jnp.float32),
            pltpu.VMEM((HALF, K), jnp.int32),
            pltpu.VMEM((HALF, K), jnp.int32),
            [pltpu.VMEM((K, width), jnp.float32) for _ in range(NBUF)],
            [pltpu.VMEM((K,), jnp.float32) for _ in range(NBUF)],
            [pltpu.SemaphoreType.DMA for _ in range(NBUF)],
            [pltpu.SemaphoreType.DMA for _ in range(NBUF)],
            [pltpu.SemaphoreType.DMA for _ in range(NBUF)],
            [pltpu.SemaphoreType.DMA for _ in range(NBUF)],
        ],
    )


def _layer1_body(acc_ref, cnt_ref, x_ref, wl_ref, wr_ref, b_ref, g_ref,
                 be_ref, h_ref, inv_ref):
    cnt_full = cnt_ref[0] + cnt_ref[1]
    inv_full = 1.0 / jnp.maximum(cnt_full, 1.0)
    inv = inv_full[:N]
    agg = acc_ref[0] + acc_ref[1]
    mean = agg * inv[:, None]
    h = (jnp.dot(mean, wl_ref[...], preferred_element_type=jnp.float32)
         + jnp.dot(x_ref[...], wr_ref[...], preferred_element_type=jnp.float32)
         + b_ref[...])
    mu = jnp.mean(h, axis=0)
    var = jnp.mean((h - mu) ** 2, axis=0)
    hn = g_ref[...] * (h - mu) / jnp.sqrt(var + 1e-5) + be_ref[...]
    h_ref[...] = jnp.concatenate(
        [jnp.maximum(hn, 0.0), jnp.zeros((NF - N, H), jnp.float32)], axis=0)
    inv_ref[...] = inv_full


def _layer23_body(acc_ref, s_ref, inv_ref, h1_ref, wl_ref, wr_ref, b_ref,
                  g_ref, be_ref, w3l_ref, w3r_ref, b3_ref, out_ref):
    inv = inv_ref[...][:N]
    h1 = h1_ref[...][:N]
    agg = acc_ref[0] + acc_ref[1]
    mean = agg * inv[:, None]
    h = (jnp.dot(mean, wl_ref[...], preferred_element_type=jnp.float32)
         + jnp.dot(h1, wr_ref[...], preferred_element_type=jnp.float32)
         + b_ref[...])
    mu = jnp.mean(h, axis=0)
    var = jnp.mean((h - mu) ** 2, axis=0)
    hn = g_ref[...] * (h - mu) / jnp.sqrt(var + 1e-5) + be_ref[...]
    h2 = jnp.maximum(hn, 0.0)
    ssum = (s_ref[0] + s_ref[1])[:N]
    v1 = jnp.sum(h2 * ssum[:, None], axis=0, keepdims=True)
    v0 = jnp.sum(h2, axis=0, keepdims=True)
    state = (jnp.dot(v1, w3l_ref[...], preferred_element_type=jnp.float32)
             + jnp.dot(v0, w3r_ref[...], preferred_element_type=jnp.float32)
             ) * (1.0 / N) + b3_ref[...][None, :]
    out_ref[...] = state


def kernel(x, edge_index, W1l, W1r, b1, g1, be1, W2l, W2r, b2, g2, be2,
           W3l, W3r, b3):
    src = edge_index[0].astype(jnp.int32)
    dst = edge_index[1].astype(jnp.int32)
    npad = EPAD - E
    src_p = jnp.concatenate(
        [src, jnp.full((npad,), N, jnp.int32)]).reshape(NW, NCHUNK, K)
    dst_p = jnp.concatenate(
        [dst, jnp.zeros((npad,), jnp.int32)]).reshape(NW, NCHUNK, K)
    zrows = jnp.zeros((N, D), jnp.float32)
    zvec = jnp.zeros((NF,), jnp.float32)
    x_pad = jnp.concatenate([x, jnp.zeros((NF - N, D), jnp.float32)])
    wtab1 = jnp.concatenate([jnp.ones((N,), jnp.float32),
                             jnp.zeros((NF - N,), jnp.float32)])

    agg1, cnt = _sc_aggregate(True, D)(x_pad, src_p, dst_p, wtab1,
                                       zrows, zvec)
    h1p, invf = pl.pallas_call(
        _layer1_body,
        out_shape=[
            jax.ShapeDtypeStruct((NF, H), jnp.float32),
            jax.ShapeDtypeStruct((NF,), jnp.float32),
        ],
    )(agg1, cnt, x, W1l, W1r, b1, g1, be1)

    agg2, svec = _sc_aggregate(False, H)(h1p, src_p, dst_p, invf,
                                         zrows, zvec)
    state = pl.pallas_call(
        _layer23_body,
        out_shape=jax.ShapeDtypeStruct((1, O), jnp.float32),
    )(agg2, svec, invf, h1p, W2l, W2r, b2, g2, be2, W3l, W3r, b3)
    return state.reshape(O)


# speedup vs baseline: 2.9179x; 2.9179x over previous
import jax
import jax.numpy as jnp
from jax import lax
from jax.experimental import pallas as pl
from jax.experimental.pallas import tpu as pltpu
from jax.experimental.pallas import tpu_sc as plsc

N = 10000
E = 320000
D = 128
H = 128
O = 64

NC = 2
NS = 16
NW = NC * NS
K = 80
NCHUNK = 128
HALF = NCHUNK // 2
EPT = NCHUNK * K
EPAD = NW * EPT
NBUF = 2
MAIN_ITERS = HALF // NBUF - 1
PADF = 2048
NF = N + PADF
ZROWS = 624
ZTAIL = N - NS * ZROWS


def _sc_aggregate(w_by_src: bool, width: int):
    mesh = plsc.VectorSubcoreMesh(core_axis_name="c", subcore_axis_name="s")

    def body(feat_hbm, src_hbm, dst_hbm, wtab_hbm, zrows_hbm, zvec_hbm,
             acc_out, side_out,
             acc_sh, side_sh, sidx, didx, rows, wvec, gsem, wgsem, ssem,
             wssem):
        c = lax.axis_index("c")
        s = lax.axis_index("s")
        tid = c * NS + s

        pltpu.sync_copy(zrows_hbm.at[pl.ds(s * ZROWS, ZROWS)],
                        acc_sh.at[pl.ds(s * ZROWS, ZROWS)])

        @pl.when(s == 0)
        def _():
            pltpu.sync_copy(zrows_hbm.at[pl.ds(NS * ZROWS, ZTAIL)],
                            acc_sh.at[pl.ds(NS * ZROWS, ZTAIL)])
            pltpu.sync_copy(zvec_hbm, side_sh)

        plsc.subcore_barrier()

        def issue_gather(b, i):
            pltpu.async_copy(feat_hbm.at[sidx.at[i]], rows[b], gsem[b])
            widx = sidx if w_by_src else didx
            pltpu.async_copy(wtab_hbm.at[widx.at[i]], wvec[b], wgsem[b])

        def wait_gather(b, i):
            pltpu.make_async_copy(feat_hbm.at[sidx.at[i]], rows[b],
                                  gsem[b]).wait()
            widx = sidx if w_by_src else didx
            pltpu.make_async_copy(wtab_hbm.at[widx.at[i]], wvec[b],
                                  wgsem[b]).wait()

        def process(b, i):
            rdesc = pltpu.async_copy(rows[b], acc_sh.at[didx.at[i]], ssem[b],
                                     add=True)
            tgt = didx if w_by_src else sidx
            wdesc = pltpu.async_copy(wvec[b], side_sh.at[tgt.at[i]], wssem[b],
                                     add=True)
            rdesc.wait()
            wdesc.wait()

        for half in range(2):
            pltpu.sync_copy(src_hbm.at[tid, pl.ds(half * HALF, HALF)], sidx)
            pltpu.sync_copy(dst_hbm.at[tid, pl.ds(half * HALF, HALF)], didx)

            for b in range(NBUF):
                issue_gather(b, b)

            def step(j, carry):
                for b in range(NBUF):
                    i = j * NBUF + b
                    wait_gather(b, i)
                    process(b, i)
                    issue_gather(b, i + NBUF)
                return carry

            lax.fori_loop(0, MAIN_ITERS, step, 0)

            for b in range(NBUF):
                i = MAIN_ITERS * NBUF + b
                wait_gather(b, i)
                process(b, i)

        plsc.subcore_barrier()

        pltpu.sync_copy(acc_sh.at[pl.ds(s * ZROWS, ZROWS)],
                        acc_out.at[c, pl.ds(s * ZROWS, ZROWS)])

        @pl.when(s == 0)
        def _():
            pltpu.sync_copy(acc_sh.at[pl.ds(NS * ZROWS, ZTAIL)],
                            acc_out.at[c, pl.ds(NS * ZROWS, ZTAIL)])
            pltpu.sync_copy(side_sh, side_out.at[c])

    return pl.kernel(
        body,
        out_type=[
            jax.ShapeDtypeStruct((NC, N, width), jnp.float32),
            jax.ShapeDtypeStruct((NC, NF), jnp.float32),
        ],
        mesh=mesh,
        scratch_types=[
            pltpu.VMEM_SHARED((N, width), jnp.float32),
            pltpu.VMEM_SHARED((NF,), jnp.float32),
            pltpu.VMEM((HALF, K), jnp.int32),
            pltpu.VMEM((HALF, K), jnp.int32),
            [pltpu.VMEM((K, width), jnp.float32) for _ in range(NBUF)],
            [pltpu.VMEM((K,), jnp.float32) for _ in range(NBUF)],
            [pltpu.SemaphoreType.DMA for _ in range(NBUF)],
            [pltpu.SemaphoreType.DMA for _ in range(NBUF)],
            [pltpu.SemaphoreType.DMA for _ in range(NBUF)],
            [pltpu.SemaphoreType.DMA for _ in range(NBUF)],
        ],
    )


def _layer1_body(acc_ref, cnt_ref, x_ref, wl_ref, wr_ref, b_ref, g_ref,
                 be_ref, h_ref, inv_ref):
    cnt_full = cnt_ref[0] + cnt_ref[1]
    inv_full = 1.0 / jnp.maximum(cnt_full, 1.0)
    inv = inv_full[:N]
    agg = acc_ref[0] + acc_ref[1]
    mean = agg * inv[:, None]
    h = (jnp.dot(mean, wl_ref[...], preferred_element_type=jnp.float32)
         + jnp.dot(x_ref[...], wr_ref[...], preferred_element_type=jnp.float32)
         + b_ref[...])
    mu = jnp.mean(h, axis=0)
    var = jnp.mean((h - mu) ** 2, axis=0)
    hn = g_ref[...] * (h - mu) / jnp.sqrt(var + 1e-5) + be_ref[...]
    h_ref[...] = jnp.concatenate(
        [jnp.maximum(hn, 0.0), jnp.zeros((NF - N, H), jnp.float32)], axis=0)
    inv_ref[...] = inv_full


def _layer23_body(acc_ref, s_ref, inv_ref, h1_ref, wl_ref, wr_ref, b_ref,
                  g_ref, be_ref, w3l_ref, w3r_ref, b3_ref, out_ref):
    inv = inv_ref[...][:N]
    h1 = h1_ref[...][:N]
    agg = acc_ref[0] + acc_ref[1]
    mean = agg * inv[:, None]
    h = (jnp.dot(mean, wl_ref[...], preferred_element_type=jnp.float32)
         + jnp.dot(h1, wr_ref[...], preferred_element_type=jnp.float32)
         + b_ref[...])
    mu = jnp.mean(h, axis=0)
    var = jnp.mean((h - mu) ** 2, axis=0)
    hn = g_ref[...] * (h - mu) / jnp.sqrt(var + 1e-5) + be_ref[...]
    h2 = jnp.maximum(hn, 0.0)
    ssum = (s_ref[0] + s_ref[1])[:N]
    v1 = jnp.sum(h2 * ssum[:, None], axis=0, keepdims=True)
    v0 = jnp.sum(h2, axis=0, keepdims=True)
    state = (jnp.dot(v1, w3l_ref[...], preferred_element_type=jnp.float32)
             + jnp.dot(v0, w3r_ref[...], preferred_element_type=jnp.float32)
             ) * (1.0 / N) + b3_ref[...][None, :]
    out_ref[...] = state


def kernel(x, edge_index, W1l, W1r, b1, g1, be1, W2l, W2r, b2, g2, be2,
           W3l, W3r, b3):
    src = edge_index[0].astype(jnp.int32)
    dst = edge_index[1].astype(jnp.int32)
    npad = EPAD - E
    pad_ids = jnp.arange(npad, dtype=jnp.int32)
    src_p = jnp.concatenate(
        [src, N + pad_ids % PADF]).reshape(NW, NCHUNK, K)
    dst_p = jnp.concatenate(
        [dst, pad_ids % N]).reshape(NW, NCHUNK, K)
    zrows = jnp.zeros((N, D), jnp.float32)
    zvec = jnp.zeros((NF,), jnp.float32)
    x_pad = jnp.concatenate([x, jnp.zeros((NF - N, D), jnp.float32)])
    wtab1 = jnp.concatenate([jnp.ones((N,), jnp.float32),
                             jnp.zeros((NF - N,), jnp.float32)])

    agg1, cnt = _sc_aggregate(True, D)(x_pad, src_p, dst_p, wtab1,
                                       zrows, zvec)
    h1p, invf = pl.pallas_call(
        _layer1_body,
        out_shape=[
            jax.ShapeDtypeStruct((NF, H), jnp.float32),
            jax.ShapeDtypeStruct((NF,), jnp.float32),
        ],
    )(agg1, cnt, x, W1l, W1r, b1, g1, be1)

    agg2, svec = _sc_aggregate(False, H)(h1p, src_p, dst_p, invf,
                                         zrows, zvec)
    state = pl.pallas_call(
        _layer23_body,
        out_shape=jax.ShapeDtypeStruct((1, O), jnp.float32),
    )(agg2, svec, invf, h1p, W2l, W2r, b2, g2, be2, W3l, W3r, b3)
    return state.reshape(O)
